# Initial kernel scaffold; baseline (speedup 1.0000x reference)
import jax
import jax.numpy as jnp
from jax.experimental import pallas as pl


def kernel(inputs):
    raise NotImplementedError("write your pallas kernel here")


# speedup vs baseline: 2.4325x; 2.4325x over previous
import functools

import jax
import jax.numpy as jnp
from jax.experimental import pallas as pl

P = 0.2
SCALE = 1.0 / (1.0 - P)
ONE_BITS = 0x3F800000


def _select_body(drop_num, n_iters, bits_row_ref, bits_col_ref, scale_ref):
    B = bits_row_ref.shape[0]
    for b in range(B):
        row = bits_row_ref[b]

        def it(_, lohi):
            lo, hi = lohi
            mid = jax.lax.div(lo + hi, 2)
            cnt = jnp.sum(jnp.where(row <= mid, 1, 0))
            ge = cnt >= drop_num
            return (jnp.where(ge, lo, mid + 1), jnp.where(ge, mid, hi))

        _, hi = jax.lax.fori_loop(
            0, n_iters, it, (jnp.int32(0), jnp.int32(ONE_BITS)))
        scale_ref[b] = jnp.where(bits_col_ref[b] <= hi,
                                 jnp.float32(0.0), jnp.float32(SCALE))


def _mul_body(x_ref, s_ref, o_ref):
    o_ref[...] = x_ref[...] * s_ref[...]


@jax.jit
def kernel(inputs):
    B, C, D = inputs.shape
    drop_num = int(round(P * C))

    rand = jax.random.uniform(jax.random.key(42), (B, C), dtype=jnp.float32)
    bits_row = jax.lax.bitcast_convert_type(rand, jnp.int32)
    bits_col = bits_row[:, :, None]

    if drop_num <= 0:
        scale = jnp.full((B, C, 1), SCALE, dtype=jnp.float32)
    else:
        scale = pl.pallas_call(
            functools.partial(_select_body, drop_num, 31),
            out_shape=jax.ShapeDtypeStruct((B, C, 1), jnp.float32),
        )(bits_row, bits_col)

    R = 512
    grid = (B, C // R)
    return pl.pallas_call(
        _mul_body,
        grid=grid,
        in_specs=[
            pl.BlockSpec((1, R, D), lambda b, c: (b, c, 0)),
            pl.BlockSpec((1, R, 1), lambda b, c: (b, c, 0)),
        ],
        out_specs=pl.BlockSpec((1, R, D), lambda b, c: (b, c, 0)),
        out_shape=jax.ShapeDtypeStruct((B, C, D), jnp.float32),
    )(inputs, scale)


# speedup vs baseline: 3.3248x; 1.3668x over previous
import functools

import jax
import jax.numpy as jnp
from jax.experimental import pallas as pl
from jax.experimental.pallas import tpu as pltpu

P = 0.2
SCALE = 1.0 / (1.0 - P)
ONE_BITS = 0x3F800000


def _fused_body(drop_num, n_iters, R,
                bits_full_ref, bits3_ref, x_ref, o_ref, thresh_ref):
    b = pl.program_id(0)
    ci = pl.program_id(1)
    B = bits_full_ref.shape[0]

    @pl.when((b == 0) & (ci == 0))
    def _init():
        for bb in range(B):
            row = bits_full_ref[bb]

            def it(_, lohi):
                lo, hi = lohi
                mid = jax.lax.div(lo + hi, 2)
                cnt = jnp.sum(jnp.where(row <= mid, 1, 0))
                ge = cnt >= drop_num
                return (jnp.where(ge, lo, mid + 1), jnp.where(ge, mid, hi))

            _, hi = jax.lax.fori_loop(
                0, n_iters, it, (jnp.int32(0), jnp.int32(ONE_BITS)))
            thresh_ref[bb] = hi

    t = thresh_ref[b]
    srow = jnp.where(bits3_ref[0] <= t, jnp.float32(0.0), jnp.float32(SCALE))
    scol = jnp.transpose(srow, (1, 0))
    o_ref[...] = x_ref[...] * scol[None]


@jax.jit
def kernel(inputs):
    B, C, D = inputs.shape
    drop_num = int(round(P * C))

    rand = jax.random.uniform(jax.random.key(42), (B, C), dtype=jnp.float32)
    bits = jax.lax.bitcast_convert_type(rand, jnp.int32)

    R = 512
    NC = C // R
    bits3 = bits.reshape(B * NC, 1, R)

    return pl.pallas_call(
        functools.partial(_fused_body, drop_num, 31, R),
        grid=(B, NC),
        in_specs=[
            pl.BlockSpec((B, C), lambda b, c: (0, 0)),
            pl.BlockSpec((1, 1, R), lambda b, c: (b * NC + c, 0, 0)),
            pl.BlockSpec((1, R, D), lambda b, c: (b, c, 0)),
        ],
        out_specs=pl.BlockSpec((1, R, D), lambda b, c: (b, c, 0)),
        out_shape=jax.ShapeDtypeStruct((B, C, D), jnp.float32),
        scratch_shapes=[pltpu.SMEM((B,), jnp.int32)],
    )(bits, bits3, inputs)


# speedup vs baseline: 3.4060x; 1.0244x over previous
import functools

import jax
import jax.numpy as jnp
from jax.experimental import pallas as pl
from jax.experimental.pallas import tpu as pltpu

P = 0.2
SCALE = 1.0 / (1.0 - P)
ONE_BITS = 0x3F800000


def _fused_body(drop_num, n_iters, R,
                bits_full_ref, bits3_ref, x_ref, o_ref, thresh_ref):
    b = pl.program_id(0)
    ci = pl.program_id(1)
    B = bits_full_ref.shape[0]

    @pl.when((b == 0) & (ci == 0))
    def _init():
        for bb in range(B):
            row = bits_full_ref[bb]

            def it(_, lohi):
                lo, hi = lohi
                mid = jax.lax.div(lo + hi, 2)
                cnt = jnp.sum(jnp.where(row <= mid, 1, 0))
                ge = cnt >= drop_num
                return (jnp.where(ge, lo, mid + 1), jnp.where(ge, mid, hi))

            _, hi = jax.lax.fori_loop(
                0, n_iters, it, (jnp.int32(0), jnp.int32(ONE_BITS)))
            thresh_ref[bb] = hi

    t = thresh_ref[b]
    srow = jnp.where(bits3_ref[0] <= t, jnp.float32(0.0), jnp.float32(SCALE))
    scol = jnp.transpose(srow, (1, 0))
    o_ref[...] = x_ref[...] * scol[None]


@jax.jit
def kernel(inputs):
    B, C, D = inputs.shape
    drop_num = int(round(P * C))

    rand = jax.random.uniform(jax.random.key(42), (B, C), dtype=jnp.float32)
    bits = jax.lax.bitcast_convert_type(rand, jnp.int32)

    R = 1024
    NC = C // R
    bits3 = bits.reshape(B * NC, 1, R)

    return pl.pallas_call(
        functools.partial(_fused_body, drop_num, 31, R),
        grid=(B, NC),
        in_specs=[
            pl.BlockSpec((B, C), lambda b, c: (0, 0)),
            pl.BlockSpec((1, 1, R), lambda b, c: (b * NC + c, 0, 0)),
            pl.BlockSpec((1, R, D), lambda b, c: (b, c, 0)),
        ],
        out_specs=pl.BlockSpec((1, R, D), lambda b, c: (b, c, 0)),
        out_shape=jax.ShapeDtypeStruct((B, C, D), jnp.float32),
        scratch_shapes=[pltpu.SMEM((B,), jnp.int32)],
    )(bits, bits3, inputs)
